# baseline (speedup 1.0000x reference)
import jax
import jax.numpy as jnp
from jax import lax
from jax.experimental import pallas as pl
from jax.experimental.pallas import tpu as pltpu
from jax.experimental.pallas import tpu_sc as plsc

NC = 2
NS = 16
NW = NC * NS

BATCH = 16384
EMBED = 64
PITCH = 2 * EMBED
CROWS = 1000
QROWS = 1000000
BPW = BATCH // NW
CHUNK = 128
NCH = BPW // CHUNK
LANES = 16
KV = EMBED // LANES


def _make_kernel():
  mesh = plsc.VectorSubcoreMesh(core_axis_name="c", subcore_axis_name="s")

  @pl.kernel(
      out_type=jax.ShapeDtypeStruct((BATCH, PITCH), jnp.float32),
      mesh=mesh,
      scratch_types=[
          pltpu.VMEM((2 * NCH, CHUNK), jnp.int32),
          pltpu.VMEM((2 * NCH, CHUNK), jnp.int32),
          pltpu.VMEM((2, CHUNK, PITCH), jnp.float32),
          pltpu.VMEM((2, CHUNK, PITCH), jnp.float32),
          pltpu.VMEM((2, CHUNK, PITCH), jnp.float32),
          pltpu.SemaphoreType.DMA,
          pltpu.SemaphoreType.DMA,
      ],
  )
  def k(idx_hbm, hidx_hbm, cpk_hbm, qpk_hbm, out_hbm,
        idx_v, hidx_v, bufc, bufq, mix, gsem, osem):
    wid = lax.axis_index("s") * NC + lax.axis_index("c")
    base = wid * BPW

    pltpu.sync_copy(idx_hbm.at[wid], idx_v)
    pltpu.sync_copy(hidx_hbm.at[wid], hidx_v)

    def issue(g, slot):
      pltpu.async_copy(cpk_hbm.at[hidx_v.at[g]], bufc.at[slot], gsem)
      pltpu.async_copy(qpk_hbm.at[hidx_v.at[NCH + g]], bufq.at[slot], gsem)

    issue(0, 0)

    def body(g, _):
      slot = g % 2

      @pl.when(g + 1 < NCH)
      def _():
        issue(g + 1, (g + 1) % 2)

      pltpu.make_async_copy(cpk_hbm.at[hidx_v.at[0]], bufc.at[slot], gsem).wait()
      pltpu.make_async_copy(cpk_hbm.at[hidx_v.at[0]], bufq.at[slot], gsem).wait()

      for rv in range(CHUNK // LANES):
        cvec = idx_v[g, pl.ds(rv * LANES, LANES)]
        qvec = idx_v[NCH + g, pl.ds(rv * LANES, LANES)]
        for l in range(LANES):
          r = rv * LANES + l
          ch = (cvec[l] & 1) * EMBED
          qh = (qvec[l] & 1) * EMBED
          for t in range(KV):
            mix[slot, r, pl.ds(t * LANES, LANES)] = bufc[
                slot, r, pl.ds(ch + t * LANES, LANES)
            ]
            mix[slot, r, pl.ds(EMBED + t * LANES, LANES)] = bufq[
                slot, r, pl.ds(qh + t * LANES, LANES)
            ]

      @pl.when(g >= 2)
      def _():
        pltpu.make_async_copy(
            mix.at[slot], out_hbm.at[pl.ds(base, CHUNK)], osem
        ).wait()

      pltpu.async_copy(
          mix.at[slot], out_hbm.at[pl.ds(base + g * CHUNK, CHUNK)], osem
      )
      return 0

    lax.fori_loop(0, NCH, body, 0)
    for _ in range(2):
      pltpu.make_async_copy(
          mix.at[0], out_hbm.at[pl.ds(base, CHUNK)], osem
      ).wait()

  return k


_kernel = _make_kernel()


@jax.jit
def kernel(x, color_table, question_table):
  xc = x[:, 0].astype(jnp.int32).reshape(NW, NCH, CHUNK)
  xq = x[:, 1].astype(jnp.int32).reshape(NW, NCH, CHUNK)
  idx_all = jnp.concatenate([xc, xq], axis=1)
  cpk = color_table.reshape(CROWS // 2, PITCH)
  qpk = question_table.reshape(QROWS // 2, PITCH)
  return _kernel(idx_all, idx_all >> 1, cpk, qpk)


# speedup vs baseline: 1.7610x; 1.7610x over previous
import jax
import jax.numpy as jnp
from jax import lax
from jax.experimental import pallas as pl
from jax.experimental.pallas import tpu as pltpu
from jax.experimental.pallas import tpu_sc as plsc

NC = 2
NS = 16
NW = NC * NS

BATCH = 16384
EMBED = 64
CROWS = 1000
QROWS = 1000000
BPW = BATCH // NW
BLK = 128
NBLK = BPW // BLK
LANES = 16


def _make_kernel():
  mesh = plsc.VectorSubcoreMesh(core_axis_name="c", subcore_axis_name="s")

  @pl.kernel(
      out_type=jax.ShapeDtypeStruct((BATCH, 2 * EMBED), jnp.float32),
      mesh=mesh,
      scratch_types=[
          pltpu.VMEM((2, BPW), jnp.int32),
          pltpu.VMEM((BPW, 2 * EMBED), jnp.float32),
          [pltpu.SemaphoreType.DMA] * NBLK,
          pltpu.SemaphoreType.DMA,
      ],
  )
  def k(idx_hbm, ctab_hbm, qtab_hbm, out_hbm, idx_s, mix, gsems, osem):
    wid = lax.axis_index("s") * NC + lax.axis_index("c")
    base = wid * BPW
    ctab3 = ctab_hbm.reshape(CROWS // 8, 8, EMBED)
    qtab3 = qtab_hbm.reshape(QROWS // 8, 8, EMBED)

    pltpu.sync_copy(idx_hbm.at[wid], idx_s)

    def issue_block(b):
      sem = gsems[b]

      def vec_group(v, _):
        r0 = b * BLK + v * LANES
        cvec = idx_s[0, pl.ds(r0, LANES)]
        qvec = idx_s[1, pl.ds(r0, LANES)]
        for j in range(LANES):
          c = cvec[j]
          q = qvec[j]
          pltpu.async_copy(
              ctab3.at[c >> 3, c & 7],
              mix.at[r0 + j, pl.ds(0, EMBED)],
              sem,
          )
          pltpu.async_copy(
              qtab3.at[q >> 3, q & 7],
              mix.at[r0 + j, pl.ds(EMBED, EMBED)],
              sem,
          )
        return 0

      lax.fori_loop(0, BLK // LANES, vec_group, 0)

    def drain_block(b):
      def row(r, _):
        for _i in range(2):
          pltpu.make_async_copy(
              ctab3.at[0, 0], mix.at[r, pl.ds(0, EMBED)], gsems[b]
          ).wait()
        return 0

      lax.fori_loop(0, BLK, row, 0)

    def write_block(b):
      pltpu.async_copy(
          mix.at[pl.ds(b * BLK, BLK)],
          out_hbm.at[pl.ds(base + b * BLK, BLK)],
          osem,
      )

    issue_block(0)
    issue_block(1)
    drain_block(0)
    write_block(0)
    issue_block(2)
    drain_block(1)
    write_block(1)
    issue_block(3)
    drain_block(2)
    write_block(2)
    drain_block(3)
    write_block(3)
    for _ in range(NBLK):
      pltpu.make_async_copy(
          mix.at[pl.ds(0, BLK)], out_hbm.at[pl.ds(base, BLK)], osem
      ).wait()

  return k


_kernel = _make_kernel()


@jax.jit
def kernel(x, color_table, question_table):
  xi = x.astype(jnp.int32).T.reshape(2, NW, BPW).transpose(1, 0, 2)
  return _kernel(xi, color_table, question_table)
